# baseline (speedup 1.0000x reference)
import functools

import jax
import jax.numpy as jnp
from jax import lax
from jax.experimental import pallas as pl
from jax.experimental.pallas import tpu as pltpu
from jax.experimental.pallas import tpu_sc as plsc

N_NODES = 10000
HIDDEN = 128
EDGE_DIM = 64
HID_I32 = HIDDEN // 2

NC, NS = 2, 16
NW = NC * NS
CHUNK = 40

EDGE_BLOCK = 16000
N_SLICES = 2


def _sc_gather(table, idx):
    n_pad = table.shape[0]
    e_pad = idx.shape[0]
    per_w = e_pad // NW
    n_chunks = per_w // CHUNK
    mesh = plsc.VectorSubcoreMesh(core_axis_name="c", subcore_axis_name="s")

    @functools.partial(
        pl.kernel,
        out_type=jax.ShapeDtypeStruct((e_pad, HIDDEN), jnp.float32),
        mesh=mesh,
        scratch_types=[
            pltpu.VMEM((per_w,), jnp.int32),
            pltpu.VMEM((CHUNK, HIDDEN), jnp.float32),
            pltpu.VMEM_SHARED((n_pad, HIDDEN), jnp.float32),
            pltpu.SemaphoreType.DMA,
        ],
    )
    def gather_kernel(table_hbm, idx_hbm, out_hbm, idx_v, rows_v, spm, sem):
        sid = lax.axis_index("s")
        wid = sid * NC + lax.axis_index("c")
        base = wid * per_w
        rows_per_sub = n_pad // NS
        pltpu.sync_copy(table_hbm.at[pl.ds(sid * rows_per_sub, rows_per_sub)],
                        spm.at[pl.ds(sid * rows_per_sub, rows_per_sub)])
        pltpu.sync_copy(idx_hbm.at[pl.ds(base, per_w)], idx_v)
        plsc.subcore_barrier()

        def body(j, _):
            pltpu.async_copy(
                spm.at[idx_v.at[pl.ds(j * CHUNK, CHUNK)]],
                rows_v, sem).wait()
            pltpu.sync_copy(rows_v, out_hbm.at[pl.ds(base + j * CHUNK, CHUNK)])
            return 0

        lax.fori_loop(0, n_chunks, body, 0)

    return gather_kernel(table, idx)


def _tc_linear_slice(g, edge_embedding, w_top, w_bot, b, n_edges,
                     block_off, prev=None):
    slice_blocks = g.shape[0] // EDGE_BLOCK

    def body(*refs):
        if prev is None:
            g_ref, e_ref, wt_ref, wb_ref, b_ref, o_ref = refs
        else:
            _, g_ref, e_ref, wt_ref, wb_ref, b_ref, o_ref = refs
        gf = g_ref[...].astype(jnp.float32)
        acc = jnp.dot(gf, wt_ref[...], preferred_element_type=jnp.float32)
        acc = acc + jnp.dot(e_ref[...], wb_ref[...],
                            preferred_element_type=jnp.float32)
        o_ref[...] = jnp.maximum(acc + b_ref[...], 0.0)

    in_specs = [
        pl.BlockSpec((EDGE_BLOCK, HIDDEN), lambda i: (i, 0)),
        pl.BlockSpec((EDGE_BLOCK, EDGE_DIM), lambda i: (i + block_off, 0)),
        pl.BlockSpec((HIDDEN, HIDDEN), lambda i: (0, 0)),
        pl.BlockSpec((EDGE_DIM, HIDDEN), lambda i: (0, 0)),
        pl.BlockSpec((1, HIDDEN), lambda i: (0, 0)),
    ]
    args = (g, edge_embedding, w_top, w_bot, b)
    aliases = {}
    if prev is not None:
        in_specs = [pl.BlockSpec((8, HIDDEN), lambda i: (0, 0))] + in_specs
        args = (prev,) + args
        aliases = {0: 0}

    return pl.pallas_call(
        body,
        grid=(slice_blocks,),
        in_specs=in_specs,
        out_specs=pl.BlockSpec((EDGE_BLOCK, HIDDEN),
                               lambda i: (i + block_off, 0)),
        out_shape=jax.ShapeDtypeStruct((n_edges, HIDDEN), jnp.float32),
        input_output_aliases=aliases,
    )(*args)


def kernel(atom_embedding, edge_index, edge_embedding, W, b):
    n_edges = edge_index.shape[1]
    src = edge_index[0]

    n_pad = ((N_NODES + 8 * NS - 1) // (8 * NS)) * (8 * NS)
    table = jnp.concatenate(
        [atom_embedding,
         jnp.zeros((n_pad - N_NODES, HIDDEN), atom_embedding.dtype)])

    sl = n_edges // N_SLICES
    assert sl % (NW * CHUNK) == 0 and sl % EDGE_BLOCK == 0

    w_top, w_bot = W[:HIDDEN], W[HIDDEN:]
    b2 = b.reshape(1, HIDDEN)

    out = None
    for k in range(N_SLICES):
        g = _sc_gather(table, lax.slice(src, (k * sl,), ((k + 1) * sl,)))
        out = _tc_linear_slice(g, edge_embedding, w_top, w_bot, b2,
                               n_edges, k * (sl // EDGE_BLOCK), prev=out)
    return out


# speedup vs baseline: 1.0461x; 1.0461x over previous
import functools

import jax
import jax.numpy as jnp
from jax import lax
from jax.experimental import pallas as pl
from jax.experimental.pallas import tpu as pltpu
from jax.experimental.pallas import tpu_sc as plsc

N_NODES = 10000
HIDDEN = 128
EDGE_DIM = 64
HID_I32 = HIDDEN // 2

NC, NS = 2, 16
NW = NC * NS
CHUNK = 80

EDGE_BLOCK = 16000
N_SLICES = 1


def _sc_gather(table, idx):
    n_rows = table.shape[0]
    e_pad = idx.shape[0]
    per_w = e_pad // NW
    n_chunks = per_w // CHUNK
    mesh = plsc.VectorSubcoreMesh(core_axis_name="c", subcore_axis_name="s")

    @functools.partial(
        pl.kernel,
        out_type=jax.ShapeDtypeStruct((e_pad, HIDDEN), jnp.float32),
        mesh=mesh,
        scratch_types=[
            pltpu.VMEM((per_w,), jnp.int32),
            pltpu.VMEM((CHUNK, HIDDEN), jnp.float32),
            pltpu.VMEM_SHARED((n_rows, HIDDEN), jnp.float32),
            pltpu.SemaphoreType.DMA,
        ],
    )
    def gather_kernel(table_hbm, idx_hbm, out_hbm, idx_v, rows_v, spm, sem):
        sid = lax.axis_index("s")
        wid = sid * NC + lax.axis_index("c")
        base = wid * per_w
        rows_per_sub = (n_rows // (8 * NS)) * 8
        tail_start = rows_per_sub * NS
        tail_rows = n_rows - tail_start
        pltpu.sync_copy(table_hbm.at[pl.ds(sid * rows_per_sub, rows_per_sub)],
                        spm.at[pl.ds(sid * rows_per_sub, rows_per_sub)])
        if tail_rows:
            pltpu.sync_copy(table_hbm.at[pl.ds(tail_start, tail_rows)],
                            spm.at[pl.ds(tail_start, tail_rows)])
        pltpu.sync_copy(idx_hbm.at[pl.ds(base, per_w)], idx_v)
        plsc.subcore_barrier()

        def body(j, _):
            pltpu.async_copy(
                spm.at[idx_v.at[pl.ds(j * CHUNK, CHUNK)]],
                rows_v, sem).wait()
            pltpu.sync_copy(rows_v, out_hbm.at[pl.ds(base + j * CHUNK, CHUNK)])
            return 0

        lax.fori_loop(0, n_chunks, body, 0)

    return gather_kernel(table, idx)


def _tc_linear_slice(g, edge_embedding, w_top, w_bot, b, n_edges,
                     block_off, prev=None):
    slice_blocks = g.shape[0] // EDGE_BLOCK

    def body(*refs):
        if prev is None:
            g_ref, e_ref, wt_ref, wb_ref, b_ref, o_ref = refs
        else:
            _, g_ref, e_ref, wt_ref, wb_ref, b_ref, o_ref = refs
        gf = g_ref[...].astype(jnp.float32)
        acc = jnp.dot(gf, wt_ref[...], preferred_element_type=jnp.float32)
        acc = acc + jnp.dot(e_ref[...], wb_ref[...],
                            preferred_element_type=jnp.float32)
        o_ref[...] = jnp.maximum(acc + b_ref[...], 0.0)

    in_specs = [
        pl.BlockSpec((EDGE_BLOCK, HIDDEN), lambda i: (i, 0)),
        pl.BlockSpec((EDGE_BLOCK, EDGE_DIM), lambda i: (i + block_off, 0)),
        pl.BlockSpec((HIDDEN, HIDDEN), lambda i: (0, 0)),
        pl.BlockSpec((EDGE_DIM, HIDDEN), lambda i: (0, 0)),
        pl.BlockSpec((1, HIDDEN), lambda i: (0, 0)),
    ]
    args = (g, edge_embedding, w_top, w_bot, b)
    aliases = {}
    if prev is not None:
        in_specs = [pl.BlockSpec((8, HIDDEN), lambda i: (0, 0))] + in_specs
        args = (prev,) + args
        aliases = {0: 0}

    return pl.pallas_call(
        body,
        grid=(slice_blocks,),
        in_specs=in_specs,
        out_specs=pl.BlockSpec((EDGE_BLOCK, HIDDEN),
                               lambda i: (i + block_off, 0)),
        out_shape=jax.ShapeDtypeStruct((n_edges, HIDDEN), jnp.float32),
        input_output_aliases=aliases,
    )(*args)


def kernel(atom_embedding, edge_index, edge_embedding, W, b):
    n_edges = edge_index.shape[1]
    src = edge_index[0]

    sl = n_edges // N_SLICES
    assert sl % (NW * CHUNK) == 0 and sl % EDGE_BLOCK == 0

    w_top, w_bot = W[:HIDDEN], W[HIDDEN:]
    b2 = b.reshape(1, HIDDEN)

    out = None
    for k in range(N_SLICES):
        g = _sc_gather(atom_embedding,
                       lax.slice(src, (k * sl,), ((k + 1) * sl,)))
        out = _tc_linear_slice(g, edge_embedding, w_top, w_bot, b2,
                               n_edges, k * (sl // EDGE_BLOCK), prev=out)
    return out


# speedup vs baseline: 1.0473x; 1.0011x over previous
import functools

import jax
import jax.numpy as jnp
from jax import lax
from jax.experimental import pallas as pl
from jax.experimental.pallas import tpu as pltpu
from jax.experimental.pallas import tpu_sc as plsc

N_NODES = 10000
HIDDEN = 128
EDGE_DIM = 64

NC, NS = 2, 16
NW = NC * NS
CHUNK = 80

EDGE_BLOCK = 16000
N_SLICES = 1


def _sc_gather(table, idx):
    n_rows = table.shape[0]
    e_pad = idx.shape[0]
    per_w = e_pad // NW
    n_chunks = per_w // CHUNK
    mesh = plsc.VectorSubcoreMesh(core_axis_name="c", subcore_axis_name="s")

    @functools.partial(
        pl.kernel,
        out_type=jax.ShapeDtypeStruct((e_pad, HIDDEN), jnp.float32),
        mesh=mesh,
        scratch_types=[
            pltpu.VMEM((per_w,), jnp.int32),
            pltpu.VMEM((CHUNK, HIDDEN), jnp.float32),
            pltpu.VMEM_SHARED((n_rows, HIDDEN), jnp.float32),
            pltpu.SemaphoreType.DMA,
        ],
    )
    def gather_kernel(table_hbm, idx_hbm, out_hbm, idx_v, rows_v, spm, sem):
        sid = lax.axis_index("s")
        wid = sid * NC + lax.axis_index("c")
        base = wid * per_w
        rows_per_sub = (n_rows // (8 * NS)) * 8
        tail_start = rows_per_sub * NS
        tail_rows = n_rows - tail_start
        pltpu.sync_copy(table_hbm.at[pl.ds(sid * rows_per_sub, rows_per_sub)],
                        spm.at[pl.ds(sid * rows_per_sub, rows_per_sub)])
        if tail_rows:
            pltpu.sync_copy(table_hbm.at[pl.ds(tail_start, tail_rows)],
                            spm.at[pl.ds(tail_start, tail_rows)])
        pltpu.sync_copy(idx_hbm.at[pl.ds(base, per_w)], idx_v)
        plsc.subcore_barrier()

        def body(j, _):
            pltpu.async_copy(
                spm.at[idx_v.at[pl.ds(j * CHUNK, CHUNK)]],
                rows_v, sem).wait()
            pltpu.sync_copy(rows_v, out_hbm.at[pl.ds(base + j * CHUNK, CHUNK)])
            return 0

        lax.fori_loop(0, n_chunks, body, 0)

    return gather_kernel(table, idx)


def _tc_linear_slice(g, edge_embedding, w_top, w_bot, b, n_edges,
                     block_off, prev=None):
    slice_blocks = g.shape[0] // EDGE_BLOCK

    def body(*refs):
        if prev is None:
            g_ref, e_ref, wt_ref, wb_ref, b_ref, o_ref = refs
        else:
            _, g_ref, e_ref, wt_ref, wb_ref, b_ref, o_ref = refs
        gf = g_ref[...].astype(jnp.float32)
        acc = jnp.dot(gf, wt_ref[...], preferred_element_type=jnp.float32)
        acc = acc + jnp.dot(e_ref[...], wb_ref[...],
                            preferred_element_type=jnp.float32)
        o_ref[...] = jnp.maximum(acc + b_ref[...], 0.0)

    in_specs = [
        pl.BlockSpec((EDGE_BLOCK, HIDDEN), lambda i: (i, 0)),
        pl.BlockSpec((EDGE_BLOCK, EDGE_DIM), lambda i: (i + block_off, 0)),
        pl.BlockSpec((HIDDEN, HIDDEN), lambda i: (0, 0)),
        pl.BlockSpec((EDGE_DIM, HIDDEN), lambda i: (0, 0)),
        pl.BlockSpec((1, HIDDEN), lambda i: (0, 0)),
    ]
    args = (g, edge_embedding, w_top, w_bot, b)
    aliases = {}
    if prev is not None:
        in_specs = [pl.BlockSpec((8, HIDDEN), lambda i: (0, 0))] + in_specs
        args = (prev,) + args
        aliases = {0: 0}

    return pl.pallas_call(
        body,
        grid=(slice_blocks,),
        in_specs=in_specs,
        out_specs=pl.BlockSpec((EDGE_BLOCK, HIDDEN),
                               lambda i: (i + block_off, 0)),
        out_shape=jax.ShapeDtypeStruct((n_edges, HIDDEN), jnp.float32),
        input_output_aliases=aliases,
    )(*args)


def kernel(atom_embedding, edge_index, edge_embedding, W, b):
    n_edges = edge_index.shape[1]
    src = edge_index[0]

    sl = n_edges // N_SLICES
    assert sl % (NW * CHUNK) == 0 and sl % EDGE_BLOCK == 0

    w_top, w_bot = W[:HIDDEN], W[HIDDEN:]
    b2 = b.reshape(1, HIDDEN)

    out = None
    for k in range(N_SLICES):
        g = _sc_gather(atom_embedding,
                       lax.slice(src, (k * sl,), ((k + 1) * sl,)))
        out = _tc_linear_slice(g, edge_embedding, w_top, w_bot, b2,
                               n_edges, k * (sl // EDGE_BLOCK), prev=out)
    return out


# speedup vs baseline: 1.0482x; 1.0009x over previous
import functools

import jax
import jax.numpy as jnp
from jax import lax
from jax.experimental import pallas as pl
from jax.experimental.pallas import tpu as pltpu
from jax.experimental.pallas import tpu_sc as plsc

N_NODES = 10000
HIDDEN = 128
EDGE_DIM = 64

NC, NS = 2, 16
NW = NC * NS
CHUNK = 80

EDGE_BLOCK = 10000
N_SLICES = 1


def _sc_gather(table, idx):
    n_rows = table.shape[0]
    e_pad = idx.shape[0]
    per_w = e_pad // NW
    n_chunks = per_w // CHUNK
    mesh = plsc.VectorSubcoreMesh(core_axis_name="c", subcore_axis_name="s")

    @functools.partial(
        pl.kernel,
        out_type=jax.ShapeDtypeStruct((e_pad, HIDDEN), jnp.float32),
        mesh=mesh,
        scratch_types=[
            pltpu.VMEM((per_w,), jnp.int32),
            pltpu.VMEM((CHUNK, HIDDEN), jnp.float32),
            pltpu.VMEM_SHARED((n_rows, HIDDEN), jnp.float32),
            pltpu.SemaphoreType.DMA,
        ],
    )
    def gather_kernel(table_hbm, idx_hbm, out_hbm, idx_v, rows_v, spm, sem):
        sid = lax.axis_index("s")
        wid = sid * NC + lax.axis_index("c")
        base = wid * per_w
        rows_per_sub = (n_rows // (8 * NS)) * 8
        tail_start = rows_per_sub * NS
        tail_rows = n_rows - tail_start
        pltpu.sync_copy(table_hbm.at[pl.ds(sid * rows_per_sub, rows_per_sub)],
                        spm.at[pl.ds(sid * rows_per_sub, rows_per_sub)])
        if tail_rows:
            pltpu.sync_copy(table_hbm.at[pl.ds(tail_start, tail_rows)],
                            spm.at[pl.ds(tail_start, tail_rows)])
        pltpu.sync_copy(idx_hbm.at[pl.ds(base, per_w)], idx_v)
        plsc.subcore_barrier()

        def body(j, _):
            pltpu.async_copy(
                spm.at[idx_v.at[pl.ds(j * CHUNK, CHUNK)]],
                rows_v, sem).wait()
            pltpu.sync_copy(rows_v, out_hbm.at[pl.ds(base + j * CHUNK, CHUNK)])
            return 0

        lax.fori_loop(0, n_chunks, body, 0)

    return gather_kernel(table, idx)


def _tc_linear_slice(g, edge_embedding, w_top, w_bot, b, n_edges,
                     block_off, prev=None):
    slice_blocks = g.shape[0] // EDGE_BLOCK

    def body(*refs):
        if prev is None:
            g_ref, e_ref, wt_ref, wb_ref, b_ref, o_ref = refs
        else:
            _, g_ref, e_ref, wt_ref, wb_ref, b_ref, o_ref = refs
        gf = g_ref[...].astype(jnp.float32)
        acc = jnp.dot(gf, wt_ref[...], preferred_element_type=jnp.float32)
        acc = acc + jnp.dot(e_ref[...], wb_ref[...],
                            preferred_element_type=jnp.float32)
        o_ref[...] = jnp.maximum(acc + b_ref[...], 0.0)

    in_specs = [
        pl.BlockSpec((EDGE_BLOCK, HIDDEN), lambda i: (i, 0)),
        pl.BlockSpec((EDGE_BLOCK, EDGE_DIM), lambda i: (i + block_off, 0)),
        pl.BlockSpec((HIDDEN, HIDDEN), lambda i: (0, 0)),
        pl.BlockSpec((EDGE_DIM, HIDDEN), lambda i: (0, 0)),
        pl.BlockSpec((1, HIDDEN), lambda i: (0, 0)),
    ]
    args = (g, edge_embedding, w_top, w_bot, b)
    aliases = {}
    if prev is not None:
        in_specs = [pl.BlockSpec((8, HIDDEN), lambda i: (0, 0))] + in_specs
        args = (prev,) + args
        aliases = {0: 0}

    return pl.pallas_call(
        body,
        grid=(slice_blocks,),
        in_specs=in_specs,
        out_specs=pl.BlockSpec((EDGE_BLOCK, HIDDEN),
                               lambda i: (i + block_off, 0)),
        out_shape=jax.ShapeDtypeStruct((n_edges, HIDDEN), jnp.float32),
        input_output_aliases=aliases,
    )(*args)


def kernel(atom_embedding, edge_index, edge_embedding, W, b):
    n_edges = edge_index.shape[1]
    src = edge_index[0]

    sl = n_edges // N_SLICES
    assert sl % (NW * CHUNK) == 0 and sl % EDGE_BLOCK == 0

    w_top, w_bot = W[:HIDDEN], W[HIDDEN:]
    b2 = b.reshape(1, HIDDEN)

    out = None
    for k in range(N_SLICES):
        g = _sc_gather(atom_embedding,
                       lax.slice(src, (k * sl,), ((k + 1) * sl,)))
        out = _tc_linear_slice(g, edge_embedding, w_top, w_bot, b2,
                               n_edges, k * (sl // EDGE_BLOCK), prev=out)
    return out
